# Initial kernel scaffold; baseline (speedup 1.0000x reference)
import jax
import jax.numpy as jnp
from jax.experimental import pallas as pl


def kernel(user_matrix, item_matrix, user_table, item_table, W1, b1, W2, b2, W3, b3, W4, b4):
    raise NotImplementedError("write your pallas kernel here")


# speedup vs baseline: 7.6695x; 7.6695x over previous
import functools

import jax
import jax.numpy as jnp
from jax import lax
from jax.experimental import pallas as pl
from jax.experimental.pallas import tpu as pltpu
from jax.experimental.pallas import tpu_sc as plsc

B, L, D = 4096, 50, 128
T = B * L
NC, NS = 2, 16
NW = NC * NS
TPW = T // NW
CH = 128
NCHUNK = TPW // CH

@functools.cache
def _get_sc_gather():
    mesh = plsc.VectorSubcoreMesh(core_axis_name="c", subcore_axis_name="s")

    @functools.partial(
        pl.kernel,
        out_type=[
            jax.ShapeDtypeStruct((T, D), jnp.float32),
            jax.ShapeDtypeStruct((T, D), jnp.float32),
        ],
        mesh=mesh,
        scratch_types=[
            pltpu.VMEM((CH,), jnp.int32),
            pltpu.VMEM((CH, D), jnp.float32),
            pltpu.SemaphoreType.DMA,
        ],
    )
    def _sc_gather(user_table, item_table, uidx, iidx, out_u, out_i,
                   idx_v, rows_v, sem):
        wid = lax.axis_index("s") * NC + lax.axis_index("c")
        base = wid * TPW

        def body(c, carry):
            off = pl.multiple_of(base + c * CH, CH)
            pltpu.sync_copy(uidx.at[pl.ds(off, CH)], idx_v)
            pltpu.async_copy(user_table.at[idx_v], rows_v, sem).wait()
            pltpu.sync_copy(rows_v, out_u.at[pl.ds(off, CH)])
            pltpu.sync_copy(iidx.at[pl.ds(off, CH)], idx_v)
            pltpu.async_copy(item_table.at[idx_v], rows_v, sem).wait()
            pltpu.sync_copy(rows_v, out_i.at[pl.ds(off, CH)])
            return carry

        lax.fori_loop(0, NCHUNK, body, 0)

    return _sc_gather


TB = 1024
GRID = T // TB


def _mlp_body(u_ref, i_ref, w1u_ref, w1i_ref, b1_ref, w2_ref, b2_ref,
              w3_ref, b3_ref, w4_ref, b4_ref, out_ref):
    u = u_ref[...].astype(jnp.bfloat16)
    it = i_ref[...].astype(jnp.bfloat16)
    h = jnp.dot(u, w1u_ref[...], preferred_element_type=jnp.float32)
    h = h + jnp.dot(it, w1i_ref[...], preferred_element_type=jnp.float32)
    h = jax.nn.relu(h + b1_ref[...])
    h = jnp.dot(h.astype(jnp.bfloat16), w2_ref[...],
                preferred_element_type=jnp.float32)
    h = jax.nn.relu(h + b2_ref[...])
    h = jnp.dot(h.astype(jnp.bfloat16), w3_ref[...],
                preferred_element_type=jnp.float32)
    h = jax.nn.relu(h + b3_ref[...])
    logit = jnp.sum(h * w4_ref[...], axis=1) + b4_ref[0, 0]
    out_ref[...] = jax.nn.sigmoid(logit)


_mlp = pl.pallas_call(
    _mlp_body,
    grid=(GRID,),
    in_specs=[
        pl.BlockSpec((TB, D), lambda g: (g, 0)),
        pl.BlockSpec((TB, D), lambda g: (g, 0)),
        pl.BlockSpec((D, 256), lambda g: (0, 0)),
        pl.BlockSpec((D, 256), lambda g: (0, 0)),
        pl.BlockSpec((1, 256), lambda g: (0, 0)),
        pl.BlockSpec((256, D), lambda g: (0, 0)),
        pl.BlockSpec((1, D), lambda g: (0, 0)),
        pl.BlockSpec((D, 64), lambda g: (0, 0)),
        pl.BlockSpec((1, 64), lambda g: (0, 0)),
        pl.BlockSpec((1, 64), lambda g: (0, 0)),
        pl.BlockSpec(memory_space=pltpu.SMEM),
    ],
    out_specs=pl.BlockSpec((TB,), lambda g: (g,)),
    out_shape=jax.ShapeDtypeStruct((T,), jnp.float32),
)


def kernel(user_matrix, item_matrix, user_table, item_table,
           W1, b1, W2, b2, W3, b3, W4, b4):
    uidx = user_matrix.reshape(-1).astype(jnp.int32)
    iidx = item_matrix.reshape(-1).astype(jnp.int32)
    u_rows, i_rows = _get_sc_gather()(user_table, item_table, uidx, iidx)

    w1t = W1.T.astype(jnp.bfloat16)
    w1u = w1t[:D]
    w1i = w1t[D:]
    w2t = W2.T.astype(jnp.bfloat16)
    w3t = W3.T.astype(jnp.bfloat16)
    w4r = W4.reshape(1, 64)
    out = _mlp(u_rows, i_rows, w1u, w1i, b1.reshape(1, 256),
               w2t, b2.reshape(1, 128), w3t, b3.reshape(1, 64),
               w4r, b4.reshape(1, 1))
    return out.reshape(B, L)


# speedup vs baseline: 8.7169x; 1.1366x over previous
import functools

import jax
import jax.numpy as jnp
from jax import lax
from jax.experimental import pallas as pl
from jax.experimental.pallas import tpu as pltpu
from jax.experimental.pallas import tpu_sc as plsc

B, L, D = 4096, 50, 128
T = B * L
NC, NS = 2, 16
NW = NC * NS
TPW = T // NW
CH = 128
NCHUNK = TPW // CH

@functools.cache
def _get_sc_gather():
    mesh = plsc.VectorSubcoreMesh(core_axis_name="c", subcore_axis_name="s")

    @functools.partial(
        pl.kernel,
        out_type=[
            jax.ShapeDtypeStruct((T, D), jnp.float32),
            jax.ShapeDtypeStruct((T, D), jnp.float32),
        ],
        mesh=mesh,
        scratch_types=[
            pltpu.VMEM((CH,), jnp.int32),
            pltpu.VMEM((CH, D), jnp.float32),
            pltpu.SemaphoreType.DMA,
        ],
    )
    def _sc_gather(user_table, item_table, uidx, iidx, out_u, out_i,
                   idx_v, rows_v, sem):
        wid = lax.axis_index("s") * NC + lax.axis_index("c")
        base = wid * TPW

        def body(c, carry):
            off = pl.multiple_of(base + c * CH, CH)
            pltpu.sync_copy(uidx.at[pl.ds(off, CH)], idx_v)
            pltpu.async_copy(user_table.at[idx_v], rows_v, sem).wait()
            pltpu.sync_copy(rows_v, out_u.at[pl.ds(off, CH)])
            pltpu.sync_copy(iidx.at[pl.ds(off, CH)], idx_v)
            pltpu.async_copy(item_table.at[idx_v], rows_v, sem).wait()
            pltpu.sync_copy(rows_v, out_i.at[pl.ds(off, CH)])
            return carry

        lax.fori_loop(0, NCHUNK, body, 0)

    return _sc_gather


TB = 1024
GRID = T // TB


_DN = (((1,), (1,)), ((), ()))


def _mlp_body(u_ref, i_ref, w1u_ref, w1i_ref, b1_ref, w2_ref, b2_ref,
              w3_ref, b3_ref, w4_ref, b4_ref, out_ref):
    u = u_ref[...].astype(jnp.bfloat16)
    it = i_ref[...].astype(jnp.bfloat16)
    h = lax.dot_general(w1u_ref[...], u, _DN,
                        preferred_element_type=jnp.float32)
    h = h + lax.dot_general(w1i_ref[...], it, _DN,
                            preferred_element_type=jnp.float32)
    h = jax.nn.relu(h + b1_ref[...])
    h = jnp.dot(w2_ref[...], h.astype(jnp.bfloat16),
                preferred_element_type=jnp.float32)
    h = jax.nn.relu(h + b2_ref[...])
    h = jnp.dot(w3_ref[...], h.astype(jnp.bfloat16),
                preferred_element_type=jnp.float32)
    h = jax.nn.relu(h + b3_ref[...])
    lg = jnp.dot(w4_ref[...], h.astype(jnp.bfloat16),
                 preferred_element_type=jnp.float32)
    lg = lg[0:1] + b4_ref[0, 0]
    out_ref[...] = jax.nn.sigmoid(lg).reshape(1, 1, TB)


_mlp_specs = dict(
    in_specs=[
        pl.BlockSpec((TB, D), lambda g: (g, 0)),
        pl.BlockSpec((TB, D), lambda g: (g, 0)),
        pl.BlockSpec((256, D), lambda g: (0, 0)),
        pl.BlockSpec((256, D), lambda g: (0, 0)),
        pl.BlockSpec((256, 1), lambda g: (0, 0)),
        pl.BlockSpec((D, 256), lambda g: (0, 0)),
        pl.BlockSpec((D, 1), lambda g: (0, 0)),
        pl.BlockSpec((64, D), lambda g: (0, 0)),
        pl.BlockSpec((64, 1), lambda g: (0, 0)),
        pl.BlockSpec((8, 64), lambda g: (0, 0)),
        pl.BlockSpec(memory_space=pltpu.SMEM),
    ],
    out_specs=pl.BlockSpec((1, 1, TB), lambda g: (g, 0, 0)),
    out_shape=jax.ShapeDtypeStruct((GRID, 1, TB), jnp.float32),
)

_mlp = pl.pallas_call(_mlp_body, grid=(GRID,), **_mlp_specs)


def kernel(user_matrix, item_matrix, user_table, item_table,
           W1, b1, W2, b2, W3, b3, W4, b4):
    uidx = user_matrix.reshape(-1).astype(jnp.int32)
    iidx = item_matrix.reshape(-1).astype(jnp.int32)
    u_rows, i_rows = _get_sc_gather()(user_table, item_table, uidx, iidx)

    w1b = W1.astype(jnp.bfloat16)
    w1u = w1b[:, :D]
    w1i = w1b[:, D:]
    w2b = W2.astype(jnp.bfloat16)
    w3b = W3.astype(jnp.bfloat16)
    w4b = jnp.broadcast_to(W4, (8, 64)).astype(jnp.bfloat16)
    out = _mlp(u_rows, i_rows, w1u, w1i, b1.reshape(256, 1),
               w2b, b2.reshape(D, 1), w3b, b3.reshape(64, 1),
               w4b, b4.reshape(1, 1))
    return out.reshape(B, L)


# speedup vs baseline: 10.8665x; 1.2466x over previous
import functools

import jax
import jax.numpy as jnp
from jax import lax
from jax.experimental import pallas as pl
from jax.experimental.pallas import tpu as pltpu
from jax.experimental.pallas import tpu_sc as plsc

B, L, D = 4096, 50, 128
T = B * L
NC, NS = 2, 16
NW = NC * NS
TPW = T // NW
CH = 128
NCHUNK = TPW // CH

@functools.cache
def _get_sc_gather():
    mesh = plsc.VectorSubcoreMesh(core_axis_name="c", subcore_axis_name="s")

    @functools.partial(
        pl.kernel,
        out_type=[
            jax.ShapeDtypeStruct((T, D), jnp.float32),
            jax.ShapeDtypeStruct((T, D), jnp.float32),
        ],
        mesh=mesh,
        scratch_types=[
            pltpu.VMEM((CH,), jnp.int32),
            pltpu.VMEM((CH,), jnp.int32),
            pltpu.VMEM((CH, D), jnp.float32),
            pltpu.VMEM((CH, D), jnp.float32),
            pltpu.SemaphoreType.DMA,
            pltpu.SemaphoreType.DMA,
            pltpu.SemaphoreType.DMA,
            pltpu.SemaphoreType.DMA,
            pltpu.SemaphoreType.DMA,
            pltpu.SemaphoreType.DMA,
        ],
    )
    def _sc_gather(user_table, item_table, uidx, iidx, out_u, out_i,
                   idx0, idx1, rows0, rows1,
                   isem0, isem1, gsem0, gsem1, osem0, osem1):
        wid = lax.axis_index("s") * NC + lax.axis_index("c")
        base = wid * TPW
        idxs, rows = (idx0, idx1), (rows0, rows1)
        isems, gsems, osems = (isem0, isem1), (gsem0, gsem1), (osem0, osem1)

        def run_table(tbl, idx_hbm, out_hbm):
            def idx_start(p, c):
                off = jnp.minimum(base + c * CH, T - CH)
                pltpu.async_copy(idx_hbm.at[pl.ds(off, CH)], idxs[p],
                                 isems[p])

            def idx_wait(p):
                pltpu.make_async_copy(idx_hbm.at[pl.ds(0, CH)], idxs[p],
                                      isems[p]).wait()

            def g_start(p):
                pltpu.async_copy(tbl.at[idxs[p]], rows[p], gsems[p])

            def g_wait(p):
                pltpu.make_async_copy(tbl.at[idxs[p]], rows[p],
                                      gsems[p]).wait()

            def o_start(p, c):
                off = pl.multiple_of(base + c * CH, CH)
                pltpu.async_copy(rows[p], out_hbm.at[pl.ds(off, CH)],
                                 osems[p])

            def o_wait(p):
                pltpu.make_async_copy(rows[p], out_hbm.at[pl.ds(0, CH)],
                                      osems[p]).wait()

            idx_start(0, 0)
            idx_start(1, 1)
            idx_wait(0)
            g_start(0)

            def body(i, carry):
                c = 2 * i
                idx_wait(1)
                g_start(1)
                g_wait(0)
                o_start(0, c)
                idx_start(0, c + 2)
                g_wait(1)
                o_start(1, c + 1)
                idx_start(1, c + 3)
                idx_wait(0)
                o_wait(0)
                g_start(0)
                o_wait(1)
                return carry

            lax.fori_loop(0, NCHUNK // 2, body, 0)
            g_wait(0)
            idx_wait(1)

        run_table(user_table, uidx, out_u)
        run_table(item_table, iidx, out_i)

    return _sc_gather


TB = 1024
GRID = T // TB


_DN = (((1,), (1,)), ((), ()))


def _mlp_body(u_ref, i_ref, w1u_ref, w1i_ref, b1_ref, w2_ref, b2_ref,
              w3_ref, b3_ref, w4_ref, b4_ref, out_ref):
    u = u_ref[...].astype(jnp.bfloat16)
    it = i_ref[...].astype(jnp.bfloat16)
    h = lax.dot_general(w1u_ref[...], u, _DN,
                        preferred_element_type=jnp.float32)
    h = h + lax.dot_general(w1i_ref[...], it, _DN,
                            preferred_element_type=jnp.float32)
    h = jax.nn.relu(h + b1_ref[...])
    h = jnp.dot(w2_ref[...], h.astype(jnp.bfloat16),
                preferred_element_type=jnp.float32)
    h = jax.nn.relu(h + b2_ref[...])
    h = jnp.dot(w3_ref[...], h.astype(jnp.bfloat16),
                preferred_element_type=jnp.float32)
    h = jax.nn.relu(h + b3_ref[...])
    lg = jnp.dot(w4_ref[...], h.astype(jnp.bfloat16),
                 preferred_element_type=jnp.float32)
    lg = lg[0:1] + b4_ref[0, 0]
    out_ref[...] = jax.nn.sigmoid(lg).reshape(1, 1, TB)


_mlp_specs = dict(
    in_specs=[
        pl.BlockSpec((TB, D), lambda g: (g, 0)),
        pl.BlockSpec((TB, D), lambda g: (g, 0)),
        pl.BlockSpec((256, D), lambda g: (0, 0)),
        pl.BlockSpec((256, D), lambda g: (0, 0)),
        pl.BlockSpec((256, 1), lambda g: (0, 0)),
        pl.BlockSpec((D, 256), lambda g: (0, 0)),
        pl.BlockSpec((D, 1), lambda g: (0, 0)),
        pl.BlockSpec((64, D), lambda g: (0, 0)),
        pl.BlockSpec((64, 1), lambda g: (0, 0)),
        pl.BlockSpec((8, 64), lambda g: (0, 0)),
        pl.BlockSpec(memory_space=pltpu.SMEM),
    ],
    out_specs=pl.BlockSpec((1, 1, TB), lambda g: (g, 0, 0)),
    out_shape=jax.ShapeDtypeStruct((GRID, 1, TB), jnp.float32),
)

_mlp = pl.pallas_call(_mlp_body, grid=(GRID,), **_mlp_specs)


def kernel(user_matrix, item_matrix, user_table, item_table,
           W1, b1, W2, b2, W3, b3, W4, b4):
    uidx = user_matrix.reshape(-1).astype(jnp.int32)
    iidx = item_matrix.reshape(-1).astype(jnp.int32)
    u_rows, i_rows = _get_sc_gather()(user_table, item_table, uidx, iidx)

    w1b = W1.astype(jnp.bfloat16)
    w1u = w1b[:, :D]
    w1i = w1b[:, D:]
    w2b = W2.astype(jnp.bfloat16)
    w3b = W3.astype(jnp.bfloat16)
    w4b = jnp.broadcast_to(W4, (8, 64)).astype(jnp.bfloat16)
    out = _mlp(u_rows, i_rows, w1u, w1i, b1.reshape(256, 1),
               w2b, b2.reshape(D, 1), w3b, b3.reshape(64, 1),
               w4b, b4.reshape(1, 1))
    return out.reshape(B, L)


# speedup vs baseline: 13.4577x; 1.2385x over previous
import functools

import jax
import jax.numpy as jnp
from jax import lax
from jax.experimental import pallas as pl
from jax.experimental.pallas import tpu as pltpu
from jax.experimental.pallas import tpu_sc as plsc

B, L, D = 4096, 50, 128
T = B * L
NC, NS = 2, 16
NW = NC * NS
TPW = T // NW
CH = 128
NCHUNK = TPW // CH

@functools.cache
def _get_sc_gather():
    mesh = plsc.VectorSubcoreMesh(core_axis_name="c", subcore_axis_name="s")

    @functools.partial(
        pl.kernel,
        out_type=[
            jax.ShapeDtypeStruct((T, D), jnp.float32),
            jax.ShapeDtypeStruct((T, D), jnp.float32),
        ],
        mesh=mesh,
        scratch_types=[
            pltpu.VMEM((CH,), jnp.int32),
            pltpu.VMEM((CH,), jnp.int32),
            pltpu.VMEM((CH, D), jnp.float32),
            pltpu.VMEM((CH, D), jnp.float32),
            pltpu.SemaphoreType.DMA,
            pltpu.SemaphoreType.DMA,
            pltpu.SemaphoreType.DMA,
            pltpu.SemaphoreType.DMA,
            pltpu.SemaphoreType.DMA,
            pltpu.SemaphoreType.DMA,
        ],
    )
    def _sc_gather(user_table, item_table, uidx, iidx, out_u, out_i,
                   idx0, idx1, rows0, rows1,
                   isem0, isem1, gsem0, gsem1, osem0, osem1):
        wid = lax.axis_index("s") * NC + lax.axis_index("c")
        base = wid * TPW
        idxs, rows = (idx0, idx1), (rows0, rows1)
        isems, gsems, osems = (isem0, isem1), (gsem0, gsem1), (osem0, osem1)

        def run_table(tbl, idx_hbm, out_hbm):
            def idx_start(p, c):
                off = jnp.minimum(base + c * CH, T - CH)
                pltpu.async_copy(idx_hbm.at[pl.ds(off, CH)], idxs[p],
                                 isems[p])

            def idx_wait(p):
                pltpu.make_async_copy(idx_hbm.at[pl.ds(0, CH)], idxs[p],
                                      isems[p]).wait()

            def g_start(p):
                pltpu.async_copy(tbl.at[idxs[p]], rows[p], gsems[p])

            def g_wait(p):
                pltpu.make_async_copy(tbl.at[idxs[p]], rows[p],
                                      gsems[p]).wait()

            def o_start(p, c):
                off = pl.multiple_of(base + c * CH, CH)
                pltpu.async_copy(rows[p], out_hbm.at[pl.ds(off, CH)],
                                 osems[p])

            def o_wait(p):
                pltpu.make_async_copy(rows[p], out_hbm.at[pl.ds(0, CH)],
                                      osems[p]).wait()

            idx_start(0, 0)
            idx_start(1, 1)
            idx_wait(0)
            g_start(0)

            def body(i, carry):
                c = 2 * i
                idx_wait(1)
                g_start(1)
                g_wait(0)
                o_start(0, c)
                idx_start(0, c + 2)
                g_wait(1)
                o_start(1, c + 1)
                idx_start(1, c + 3)
                idx_wait(0)
                o_wait(0)
                g_start(0)
                o_wait(1)
                return carry

            lax.fori_loop(0, NCHUNK // 2, body, 0)
            g_wait(0)
            idx_wait(1)

        run_table(user_table, uidx, out_u)
        run_table(item_table, iidx, out_i)

    return _sc_gather


TB = 2048
GRID = T // TB


_DN = (((1,), (1,)), ((), ()))


def _mlp_body(u_ref, i_ref, w1u_ref, w1i_ref, b1_ref, w2_ref, b2_ref,
              w3_ref, b3_ref, w4_ref, b4_ref, out_ref):
    u = u_ref[...].astype(jnp.bfloat16)
    it = i_ref[...].astype(jnp.bfloat16)
    h = lax.dot_general(w1u_ref[...], u, _DN,
                        preferred_element_type=jnp.float32)
    h = h + lax.dot_general(w1i_ref[...], it, _DN,
                            preferred_element_type=jnp.float32)
    h = jax.nn.relu(h + b1_ref[...])
    h = jnp.dot(w2_ref[...], h.astype(jnp.bfloat16),
                preferred_element_type=jnp.float32)
    h = jax.nn.relu(h + b2_ref[...])
    h = jnp.dot(w3_ref[...], h.astype(jnp.bfloat16),
                preferred_element_type=jnp.float32)
    h = jax.nn.relu(h + b3_ref[...])
    lg = jnp.dot(w4_ref[...], h.astype(jnp.bfloat16),
                 preferred_element_type=jnp.float32)
    lg = lg[0:1] + b4_ref[0, 0]
    out_ref[...] = jax.nn.sigmoid(lg).reshape(1, 1, TB)


_mlp_specs = dict(
    in_specs=[
        pl.BlockSpec((TB, D), lambda g: (g, 0)),
        pl.BlockSpec((TB, D), lambda g: (g, 0)),
        pl.BlockSpec((256, D), lambda g: (0, 0)),
        pl.BlockSpec((256, D), lambda g: (0, 0)),
        pl.BlockSpec((256, 1), lambda g: (0, 0)),
        pl.BlockSpec((D, 256), lambda g: (0, 0)),
        pl.BlockSpec((D, 1), lambda g: (0, 0)),
        pl.BlockSpec((64, D), lambda g: (0, 0)),
        pl.BlockSpec((64, 1), lambda g: (0, 0)),
        pl.BlockSpec((8, 64), lambda g: (0, 0)),
        pl.BlockSpec(memory_space=pltpu.SMEM),
    ],
    out_specs=pl.BlockSpec((1, 1, TB), lambda g: (g, 0, 0)),
    out_shape=jax.ShapeDtypeStruct((GRID, 1, TB), jnp.float32),
)

_mlp = pl.pallas_call(_mlp_body, grid=(GRID,), **_mlp_specs)


def kernel(user_matrix, item_matrix, user_table, item_table,
           W1, b1, W2, b2, W3, b3, W4, b4):
    uidx = user_matrix.reshape(-1).astype(jnp.int32)
    iidx = item_matrix.reshape(-1).astype(jnp.int32)
    u_rows, i_rows = _get_sc_gather()(user_table, item_table, uidx, iidx)

    w1b = W1.astype(jnp.bfloat16)
    w1u = w1b[:, :D]
    w1i = w1b[:, D:]
    w2b = W2.astype(jnp.bfloat16)
    w3b = W3.astype(jnp.bfloat16)
    w4b = jnp.broadcast_to(W4, (8, 64)).astype(jnp.bfloat16)
    out = _mlp(u_rows, i_rows, w1u, w1i, b1.reshape(256, 1),
               w2b, b2.reshape(D, 1), w3b, b3.reshape(64, 1),
               w4b, b4.reshape(1, 1))
    return out.reshape(B, L)


# speedup vs baseline: 15.5592x; 1.1562x over previous
import functools

import jax
import jax.numpy as jnp
from jax import lax
from jax.experimental import pallas as pl
from jax.experimental.pallas import tpu as pltpu
from jax.experimental.pallas import tpu_sc as plsc

B, L, D = 4096, 50, 128
T = B * L
NC, NS = 2, 16
NW = NC * NS
CH = 128
S = 5
TS = T // S
TB = 2048


@functools.cache
def _get_sc_gather(ts):
    tpw = ts // NW
    nchunk = tpw // CH
    assert nchunk % 2 == 0 and tpw % CH == 0
    mesh = plsc.VectorSubcoreMesh(core_axis_name="c", subcore_axis_name="s")

    @functools.partial(
        pl.kernel,
        out_type=[
            jax.ShapeDtypeStruct((ts, D), jnp.float32),
            jax.ShapeDtypeStruct((ts, D), jnp.float32),
        ],
        mesh=mesh,
        scratch_types=[
            pltpu.VMEM((CH,), jnp.int32),
            pltpu.VMEM((CH,), jnp.int32),
            pltpu.VMEM((CH, D), jnp.float32),
            pltpu.VMEM((CH, D), jnp.float32),
            pltpu.SemaphoreType.DMA,
            pltpu.SemaphoreType.DMA,
            pltpu.SemaphoreType.DMA,
            pltpu.SemaphoreType.DMA,
            pltpu.SemaphoreType.DMA,
            pltpu.SemaphoreType.DMA,
        ],
    )
    def _sc_gather(user_table, item_table, uidx, iidx, out_u, out_i,
                   idx0, idx1, rows0, rows1,
                   isem0, isem1, gsem0, gsem1, osem0, osem1):
        wid = lax.axis_index("s") * NC + lax.axis_index("c")
        base = wid * tpw
        idxs, rows = (idx0, idx1), (rows0, rows1)
        isems, gsems, osems = (isem0, isem1), (gsem0, gsem1), (osem0, osem1)

        def run_table(tbl, idx_hbm, out_hbm):
            def idx_start(p, c):
                off = jnp.minimum(base + c * CH, ts - CH)
                pltpu.async_copy(idx_hbm.at[pl.ds(off, CH)], idxs[p],
                                 isems[p])

            def idx_wait(p):
                pltpu.make_async_copy(idx_hbm.at[pl.ds(0, CH)], idxs[p],
                                      isems[p]).wait()

            def g_start(p):
                pltpu.async_copy(tbl.at[idxs[p]], rows[p], gsems[p])

            def g_wait(p):
                pltpu.make_async_copy(tbl.at[idxs[p]], rows[p],
                                      gsems[p]).wait()

            def o_start(p, c):
                off = pl.multiple_of(base + c * CH, CH)
                pltpu.async_copy(rows[p], out_hbm.at[pl.ds(off, CH)],
                                 osems[p])

            def o_wait(p):
                pltpu.make_async_copy(rows[p], out_hbm.at[pl.ds(0, CH)],
                                      osems[p]).wait()

            idx_start(0, 0)
            idx_start(1, 1)
            idx_wait(0)
            g_start(0)

            def body(i, carry):
                c = 2 * i
                idx_wait(1)
                g_start(1)
                g_wait(0)
                o_start(0, c)
                idx_start(0, c + 2)
                g_wait(1)
                o_start(1, c + 1)
                idx_start(1, c + 3)
                idx_wait(0)
                o_wait(0)
                g_start(0)
                o_wait(1)
                return carry

            lax.fori_loop(0, nchunk // 2, body, 0)
            g_wait(0)
            idx_wait(1)

        run_table(user_table, uidx, out_u)
        run_table(item_table, iidx, out_i)

    return _sc_gather


_DN = (((1,), (1,)), ((), ()))


def _mlp_body(u_ref, i_ref, w1u_ref, w1i_ref, b1_ref, w2_ref, b2_ref,
              w3_ref, b3_ref, w4_ref, b4_ref, out_ref):
    u = u_ref[...].astype(jnp.bfloat16)
    it = i_ref[...].astype(jnp.bfloat16)
    h = lax.dot_general(w1u_ref[...], u, _DN,
                        preferred_element_type=jnp.float32)
    h = h + lax.dot_general(w1i_ref[...], it, _DN,
                            preferred_element_type=jnp.float32)
    h = jax.nn.relu(h + b1_ref[...])
    h = jnp.dot(w2_ref[...], h.astype(jnp.bfloat16),
                preferred_element_type=jnp.float32)
    h = jax.nn.relu(h + b2_ref[...])
    h = jnp.dot(w3_ref[...], h.astype(jnp.bfloat16),
                preferred_element_type=jnp.float32)
    h = jax.nn.relu(h + b3_ref[...])
    lg = jnp.dot(w4_ref[...], h.astype(jnp.bfloat16),
                 preferred_element_type=jnp.float32)
    lg = lg[0:1] + b4_ref[0, 0]
    out_ref[...] = jax.nn.sigmoid(lg).reshape(1, 1, TB)


def _mk_mlp_specs(ts):
    return dict(
        in_specs=[
            pl.BlockSpec((TB, D), lambda g: (g, 0)),
            pl.BlockSpec((TB, D), lambda g: (g, 0)),
            pl.BlockSpec((256, D), lambda g: (0, 0)),
            pl.BlockSpec((256, D), lambda g: (0, 0)),
            pl.BlockSpec((256, 1), lambda g: (0, 0)),
            pl.BlockSpec((D, 256), lambda g: (0, 0)),
            pl.BlockSpec((D, 1), lambda g: (0, 0)),
            pl.BlockSpec((64, D), lambda g: (0, 0)),
            pl.BlockSpec((64, 1), lambda g: (0, 0)),
            pl.BlockSpec((8, 64), lambda g: (0, 0)),
            pl.BlockSpec(memory_space=pltpu.SMEM),
        ],
        out_specs=pl.BlockSpec((1, 1, TB), lambda g: (g, 0, 0)),
        out_shape=jax.ShapeDtypeStruct((ts // TB, 1, TB), jnp.float32),
    )


@functools.cache
def _get_mlp(ts):
    return pl.pallas_call(_mlp_body, grid=(ts // TB,), **_mk_mlp_specs(ts))


def kernel(user_matrix, item_matrix, user_table, item_table,
           W1, b1, W2, b2, W3, b3, W4, b4):
    uidx = user_matrix.reshape(-1).astype(jnp.int32)
    iidx = item_matrix.reshape(-1).astype(jnp.int32)

    w1b = W1.astype(jnp.bfloat16)
    w1u = w1b[:, :D]
    w1i = w1b[:, D:]
    w2b = W2.astype(jnp.bfloat16)
    w3b = W3.astype(jnp.bfloat16)
    w4b = jnp.broadcast_to(W4, (8, 64)).astype(jnp.bfloat16)
    wargs = (w1u, w1i, b1.reshape(256, 1), w2b, b2.reshape(D, 1),
             w3b, b3.reshape(64, 1), w4b, b4.reshape(1, 1))

    gather = _get_sc_gather(TS)
    mlp = _get_mlp(TS)
    outs = []
    for s in range(S):
        u_rows, i_rows = gather(user_table, item_table,
                                uidx[s * TS:(s + 1) * TS],
                                iidx[s * TS:(s + 1) * TS])
        outs.append(mlp(u_rows, i_rows, *wargs))
    return jnp.concatenate(outs).reshape(B, L)


# speedup vs baseline: 15.8956x; 1.0216x over previous
import functools

import jax
import jax.numpy as jnp
from jax import lax
from jax.experimental import pallas as pl
from jax.experimental.pallas import tpu as pltpu
from jax.experimental.pallas import tpu_sc as plsc

B, L, D = 4096, 50, 128
T = B * L
NC, NS = 2, 16
NW = NC * NS
CH = 128
S = 5
TS = T // S
TB = 2048


@functools.cache
def _get_sc_gather(ts):
    tpw = ts // NW
    nchunk = tpw // CH
    assert nchunk % 2 == 0 and tpw % CH == 0
    mesh = plsc.VectorSubcoreMesh(core_axis_name="c", subcore_axis_name="s")

    @functools.partial(
        pl.kernel,
        out_type=[
            jax.ShapeDtypeStruct((ts, D), jnp.float32),
            jax.ShapeDtypeStruct((ts, D), jnp.float32),
        ],
        mesh=mesh,
        scratch_types=(
            [pltpu.VMEM((CH,), jnp.int32)] * 4
            + [pltpu.VMEM((CH, D), jnp.float32)] * 4
            + [pltpu.SemaphoreType.DMA] * 12
        ),
    )
    def _sc_gather(user_table, item_table, uidx, iidx, out_u, out_i,
                   ui0, ui1, ii0, ii1, ur0, ur1, ir0, ir1,
                   uis0, uis1, iis0, iis1, ugs0, ugs1, igs0, igs1,
                   uos0, uos1, ios0, ios1):
        wid = lax.axis_index("s") * NC + lax.axis_index("c")
        base = wid * tpw

        U = (user_table, uidx, out_u, (ui0, ui1), (ur0, ur1),
             (uis0, uis1), (ugs0, ugs1), (uos0, uos1))
        I = (item_table, iidx, out_i, (ii0, ii1), (ir0, ir1),
             (iis0, iis1), (igs0, igs1), (ios0, ios1))

        def make_ops(t):
            tbl, idx_hbm, out_hbm, idxs, rows, isems, gsems, osems = t

            def idx_start(p, c):
                off = jnp.minimum(base + c * CH, ts - CH)
                pltpu.async_copy(idx_hbm.at[pl.ds(off, CH)], idxs[p],
                                 isems[p])

            def idx_wait(p):
                pltpu.make_async_copy(idx_hbm.at[pl.ds(0, CH)], idxs[p],
                                      isems[p]).wait()

            def g_start(p):
                pltpu.async_copy(tbl.at[idxs[p]], rows[p], gsems[p])

            def g_wait(p):
                pltpu.make_async_copy(tbl.at[idxs[p]], rows[p],
                                      gsems[p]).wait()

            def o_start(p, c):
                off = pl.multiple_of(base + c * CH, CH)
                pltpu.async_copy(rows[p], out_hbm.at[pl.ds(off, CH)],
                                 osems[p])

            def o_wait(p):
                pltpu.make_async_copy(rows[p], out_hbm.at[pl.ds(0, CH)],
                                      osems[p]).wait()

            return idx_start, idx_wait, g_start, g_wait, o_start, o_wait

        u_ops = make_ops(U)
        i_ops = make_ops(I)

        for idx_start, idx_wait, g_start, g_wait, o_start, o_wait in (
                u_ops, i_ops):
            idx_start(0, 0)
            idx_start(1, 1)
        for idx_start, idx_wait, g_start, g_wait, o_start, o_wait in (
                u_ops, i_ops):
            idx_wait(0)
            g_start(0)

        def body(i, carry):
            c = 2 * i
            for idx_start, idx_wait, g_start, g_wait, o_start, o_wait in (
                    u_ops, i_ops):
                idx_wait(1)
                g_start(1)
            for idx_start, idx_wait, g_start, g_wait, o_start, o_wait in (
                    u_ops, i_ops):
                g_wait(0)
                o_start(0, c)
                idx_start(0, c + 2)
            for idx_start, idx_wait, g_start, g_wait, o_start, o_wait in (
                    u_ops, i_ops):
                g_wait(1)
                o_start(1, c + 1)
                idx_start(1, c + 3)
            for idx_start, idx_wait, g_start, g_wait, o_start, o_wait in (
                    u_ops, i_ops):
                idx_wait(0)
                o_wait(0)
                g_start(0)
            for idx_start, idx_wait, g_start, g_wait, o_start, o_wait in (
                    u_ops, i_ops):
                o_wait(1)
            return carry

        lax.fori_loop(0, nchunk // 2, body, 0)
        for idx_start, idx_wait, g_start, g_wait, o_start, o_wait in (
                u_ops, i_ops):
            g_wait(0)
            idx_wait(1)

    return _sc_gather


_DN = (((1,), (1,)), ((), ()))


def _mlp_body(u_ref, i_ref, w1u_ref, w1i_ref, b1_ref, w2_ref, b2_ref,
              w3_ref, b3_ref, w4_ref, b4_ref, out_ref):
    u = u_ref[...].astype(jnp.bfloat16)
    it = i_ref[...].astype(jnp.bfloat16)
    h = lax.dot_general(w1u_ref[...], u, _DN,
                        preferred_element_type=jnp.float32)
    h = h + lax.dot_general(w1i_ref[...], it, _DN,
                            preferred_element_type=jnp.float32)
    h = jax.nn.relu(h + b1_ref[...])
    h = jnp.dot(w2_ref[...], h.astype(jnp.bfloat16),
                preferred_element_type=jnp.float32)
    h = jax.nn.relu(h + b2_ref[...])
    h = jnp.dot(w3_ref[...], h.astype(jnp.bfloat16),
                preferred_element_type=jnp.float32)
    h = jax.nn.relu(h + b3_ref[...])
    lg = jnp.dot(w4_ref[...], h.astype(jnp.bfloat16),
                 preferred_element_type=jnp.float32)
    lg = lg[0:1] + b4_ref[0, 0]
    out_ref[...] = jax.nn.sigmoid(lg).reshape(1, 1, TB)


def _mk_mlp_specs(ts):
    return dict(
        in_specs=[
            pl.BlockSpec((TB, D), lambda g: (g, 0)),
            pl.BlockSpec((TB, D), lambda g: (g, 0)),
            pl.BlockSpec((256, D), lambda g: (0, 0)),
            pl.BlockSpec((256, D), lambda g: (0, 0)),
            pl.BlockSpec((256, 1), lambda g: (0, 0)),
            pl.BlockSpec((D, 256), lambda g: (0, 0)),
            pl.BlockSpec((D, 1), lambda g: (0, 0)),
            pl.BlockSpec((64, D), lambda g: (0, 0)),
            pl.BlockSpec((64, 1), lambda g: (0, 0)),
            pl.BlockSpec((8, 64), lambda g: (0, 0)),
            pl.BlockSpec(memory_space=pltpu.SMEM),
        ],
        out_specs=pl.BlockSpec((1, 1, TB), lambda g: (g, 0, 0)),
        out_shape=jax.ShapeDtypeStruct((ts // TB, 1, TB), jnp.float32),
    )


@functools.cache
def _get_mlp(ts):
    return pl.pallas_call(_mlp_body, grid=(ts // TB,), **_mk_mlp_specs(ts))


def kernel(user_matrix, item_matrix, user_table, item_table,
           W1, b1, W2, b2, W3, b3, W4, b4):
    uidx = user_matrix.reshape(-1).astype(jnp.int32)
    iidx = item_matrix.reshape(-1).astype(jnp.int32)

    w1b = W1.astype(jnp.bfloat16)
    w1u = w1b[:, :D]
    w1i = w1b[:, D:]
    w2b = W2.astype(jnp.bfloat16)
    w3b = W3.astype(jnp.bfloat16)
    w4b = jnp.broadcast_to(W4, (8, 64)).astype(jnp.bfloat16)
    wargs = (w1u, w1i, b1.reshape(256, 1), w2b, b2.reshape(D, 1),
             w3b, b3.reshape(64, 1), w4b, b4.reshape(1, 1))

    gather = _get_sc_gather(TS)
    mlp = _get_mlp(TS)
    outs = []
    for s in range(S):
        u_rows, i_rows = gather(user_table, item_table,
                                uidx[s * TS:(s + 1) * TS],
                                iidx[s * TS:(s + 1) * TS])
        outs.append(mlp(u_rows, i_rows, *wargs))
    return jnp.concatenate(outs).reshape(B, L)


# speedup vs baseline: 16.8057x; 1.0573x over previous
import functools

import jax
import jax.numpy as jnp
from jax import lax
from jax.experimental import pallas as pl
from jax.experimental.pallas import tpu as pltpu
from jax.experimental.pallas import tpu_sc as plsc

B, L, D = 4096, 50, 128
T = B * L
NC, NS = 2, 16
NW = NC * NS
CH = 128
S = 5
TS = T // S
TB = 4096


@functools.cache
def _get_sc_gather(ts):
    tpw = ts // NW
    nchunk = tpw // CH
    assert nchunk % 2 == 0 and tpw % CH == 0
    mesh = plsc.VectorSubcoreMesh(core_axis_name="c", subcore_axis_name="s")

    @functools.partial(
        pl.kernel,
        out_type=[
            jax.ShapeDtypeStruct((ts, D), jnp.float32),
            jax.ShapeDtypeStruct((ts, D), jnp.float32),
        ],
        mesh=mesh,
        scratch_types=(
            [pltpu.VMEM((CH,), jnp.int32)] * 4
            + [pltpu.VMEM((CH, D), jnp.float32)] * 4
            + [pltpu.SemaphoreType.DMA] * 12
        ),
    )
    def _sc_gather(user_table, item_table, uidx, iidx, out_u, out_i,
                   ui0, ui1, ii0, ii1, ur0, ur1, ir0, ir1,
                   uis0, uis1, iis0, iis1, ugs0, ugs1, igs0, igs1,
                   uos0, uos1, ios0, ios1):
        wid = lax.axis_index("s") * NC + lax.axis_index("c")
        base = wid * tpw

        U = (user_table, uidx, out_u, (ui0, ui1), (ur0, ur1),
             (uis0, uis1), (ugs0, ugs1), (uos0, uos1))
        I = (item_table, iidx, out_i, (ii0, ii1), (ir0, ir1),
             (iis0, iis1), (igs0, igs1), (ios0, ios1))

        def make_ops(t):
            tbl, idx_hbm, out_hbm, idxs, rows, isems, gsems, osems = t

            def idx_start(p, c):
                off = jnp.minimum(base + c * CH, ts - CH)
                pltpu.async_copy(idx_hbm.at[pl.ds(off, CH)], idxs[p],
                                 isems[p])

            def idx_wait(p):
                pltpu.make_async_copy(idx_hbm.at[pl.ds(0, CH)], idxs[p],
                                      isems[p]).wait()

            def g_start(p):
                pltpu.async_copy(tbl.at[idxs[p]], rows[p], gsems[p])

            def g_wait(p):
                pltpu.make_async_copy(tbl.at[idxs[p]], rows[p],
                                      gsems[p]).wait()

            def o_start(p, c):
                off = pl.multiple_of(base + c * CH, CH)
                pltpu.async_copy(rows[p], out_hbm.at[pl.ds(off, CH)],
                                 osems[p])

            def o_wait(p):
                pltpu.make_async_copy(rows[p], out_hbm.at[pl.ds(0, CH)],
                                      osems[p]).wait()

            return idx_start, idx_wait, g_start, g_wait, o_start, o_wait

        u_ops = make_ops(U)
        i_ops = make_ops(I)

        for idx_start, idx_wait, g_start, g_wait, o_start, o_wait in (
                u_ops, i_ops):
            idx_start(0, 0)
            idx_start(1, 1)
        for idx_start, idx_wait, g_start, g_wait, o_start, o_wait in (
                u_ops, i_ops):
            idx_wait(0)
            g_start(0)

        def body(i, carry):
            c = 2 * i
            for idx_start, idx_wait, g_start, g_wait, o_start, o_wait in (
                    u_ops, i_ops):
                idx_wait(1)
                g_start(1)
            for idx_start, idx_wait, g_start, g_wait, o_start, o_wait in (
                    u_ops, i_ops):
                g_wait(0)
                o_start(0, c)
                idx_start(0, c + 2)
            for idx_start, idx_wait, g_start, g_wait, o_start, o_wait in (
                    u_ops, i_ops):
                g_wait(1)
                o_start(1, c + 1)
                idx_start(1, c + 3)
            for idx_start, idx_wait, g_start, g_wait, o_start, o_wait in (
                    u_ops, i_ops):
                idx_wait(0)
                o_wait(0)
                g_start(0)
            for idx_start, idx_wait, g_start, g_wait, o_start, o_wait in (
                    u_ops, i_ops):
                o_wait(1)
            return carry

        lax.fori_loop(0, nchunk // 2, body, 0)
        for idx_start, idx_wait, g_start, g_wait, o_start, o_wait in (
                u_ops, i_ops):
            g_wait(0)
            idx_wait(1)

    return _sc_gather


_DN = (((1,), (1,)), ((), ()))


def _mlp_body(u_ref, i_ref, w1u_ref, w1i_ref, b1_ref, w2_ref, b2_ref,
              w3_ref, b3_ref, w4_ref, b4_ref, out_ref):
    u = u_ref[...].astype(jnp.bfloat16)
    it = i_ref[...].astype(jnp.bfloat16)
    h = lax.dot_general(w1u_ref[...], u, _DN,
                        preferred_element_type=jnp.float32)
    h = h + lax.dot_general(w1i_ref[...], it, _DN,
                            preferred_element_type=jnp.float32)
    h = jax.nn.relu(h + b1_ref[...])
    h = jnp.dot(w2_ref[...], h.astype(jnp.bfloat16),
                preferred_element_type=jnp.float32)
    h = jax.nn.relu(h + b2_ref[...])
    h = jnp.dot(w3_ref[...], h.astype(jnp.bfloat16),
                preferred_element_type=jnp.float32)
    h = jax.nn.relu(h + b3_ref[...])
    lg = jnp.dot(w4_ref[...], h.astype(jnp.bfloat16),
                 preferred_element_type=jnp.float32)
    lg = lg[0:1] + b4_ref[0, 0]
    out_ref[...] = jax.nn.sigmoid(lg).reshape(1, 1, TB)


def _mk_mlp_specs(ts):
    return dict(
        in_specs=[
            pl.BlockSpec((TB, D), lambda g: (g, 0)),
            pl.BlockSpec((TB, D), lambda g: (g, 0)),
            pl.BlockSpec((256, D), lambda g: (0, 0)),
            pl.BlockSpec((256, D), lambda g: (0, 0)),
            pl.BlockSpec((256, 1), lambda g: (0, 0)),
            pl.BlockSpec((D, 256), lambda g: (0, 0)),
            pl.BlockSpec((D, 1), lambda g: (0, 0)),
            pl.BlockSpec((64, D), lambda g: (0, 0)),
            pl.BlockSpec((64, 1), lambda g: (0, 0)),
            pl.BlockSpec((8, 64), lambda g: (0, 0)),
            pl.BlockSpec(memory_space=pltpu.SMEM),
        ],
        out_specs=pl.BlockSpec((1, 1, TB), lambda g: (g, 0, 0)),
        out_shape=jax.ShapeDtypeStruct((ts // TB, 1, TB), jnp.float32),
    )


@functools.cache
def _get_mlp(ts):
    return pl.pallas_call(_mlp_body, grid=(ts // TB,), **_mk_mlp_specs(ts))


def kernel(user_matrix, item_matrix, user_table, item_table,
           W1, b1, W2, b2, W3, b3, W4, b4):
    uidx = user_matrix.reshape(-1).astype(jnp.int32)
    iidx = item_matrix.reshape(-1).astype(jnp.int32)

    w1b = W1.astype(jnp.bfloat16)
    w1u = w1b[:, :D]
    w1i = w1b[:, D:]
    w2b = W2.astype(jnp.bfloat16)
    w3b = W3.astype(jnp.bfloat16)
    w4b = jnp.broadcast_to(W4, (8, 64)).astype(jnp.bfloat16)
    wargs = (w1u, w1i, b1.reshape(256, 1), w2b, b2.reshape(D, 1),
             w3b, b3.reshape(64, 1), w4b, b4.reshape(1, 1))

    gather = _get_sc_gather(TS)
    mlp = _get_mlp(TS)
    outs = []
    for s in range(S):
        u_rows, i_rows = gather(user_table, item_table,
                                uidx[s * TS:(s + 1) * TS],
                                iidx[s * TS:(s + 1) * TS])
        outs.append(mlp(u_rows, i_rows, *wargs))
    return jnp.concatenate(outs).reshape(B, L)
